# Initial kernel scaffold; baseline (speedup 1.0000x reference)
import jax
import jax.numpy as jnp
from jax.experimental import pallas as pl


def kernel(x, edge_index, W1, b1, W2, b2):
    raise NotImplementedError("write your pallas kernel here")


# speedup vs baseline: 15.9868x; 15.9868x over previous
import functools

import jax
import jax.numpy as jnp
from jax import lax
from jax.experimental import pallas as pl
from jax.experimental.pallas import tpu as pltpu
from jax.experimental.pallas import tpu_sc as plsc

N = 10000
N_PAD = 10112
E = 320000
D = 128
NC = 2
NS = 16
NW = NC * NS
CK = 80
NCHUNK = E // (NW * CK)
RPT = N_PAD // NS

_mesh = plsc.VectorSubcoreMesh(core_axis_name="c", subcore_axis_name="s")


@functools.partial(
    pl.kernel,
    out_type=jax.ShapeDtypeStruct((NC, N_PAD, 16), jnp.float32),
    mesh=_mesh,
    scratch_types=[
        pltpu.VMEM((NCHUNK, CK), jnp.int32),
        pltpu.VMEM((CK, 16), jnp.float32),
        pltpu.VMEM_SHARED((N_PAD, 16), jnp.float32),
    ],
)
def _deg_count(dst_hbm, zeros16_hbm, out_hbm, dst_v, ones_v, acc_sh):
    c = lax.axis_index("c")
    s = lax.axis_index("s")
    wid = c * NS + s
    pltpu.sync_copy(zeros16_hbm.at[pl.ds(s * RPT, RPT)],
                    acc_sh.at[pl.ds(s * RPT, RPT)])
    pltpu.sync_copy(dst_hbm.at[wid], dst_v)
    for r in range(CK):
        ones_v[r] = jnp.ones((16,), jnp.float32)
    plsc.subcore_barrier()

    def body(i, carry):
        pltpu.sync_copy(ones_v, acc_sh.at[dst_v.at[i]], add=True)
        return carry

    lax.fori_loop(0, NCHUNK, body, 0)
    plsc.subcore_barrier()
    pltpu.sync_copy(acc_sh.at[pl.ds(s * RPT, RPT)],
                    out_hbm.at[c, pl.ds(s * RPT, RPT)])


@functools.partial(
    pl.kernel,
    out_type=jax.ShapeDtypeStruct((NC, N_PAD, D), jnp.float32),
    mesh=_mesh,
    scratch_types=[
        pltpu.VMEM((NCHUNK, CK), jnp.int32),
        pltpu.VMEM((NCHUNK, CK), jnp.int32),
        pltpu.VMEM((CK, D), jnp.float32),
        pltpu.VMEM_SHARED((N_PAD, D), jnp.float32),
        pltpu.SemaphoreType.DMA,
    ],
)
def _edge_agg(g_hbm, src_hbm, dst_hbm, zeros_hbm, out_hbm,
              src_v, dst_v, rows_v, acc_sh, sem):
    c = lax.axis_index("c")
    s = lax.axis_index("s")
    wid = c * NS + s
    pltpu.sync_copy(zeros_hbm.at[pl.ds(s * RPT, RPT)],
                    acc_sh.at[pl.ds(s * RPT, RPT)])
    pltpu.sync_copy(src_hbm.at[wid], src_v)
    pltpu.sync_copy(dst_hbm.at[wid], dst_v)
    plsc.subcore_barrier()

    def body(i, carry):
        pltpu.async_copy(g_hbm.at[src_v.at[i]], rows_v, sem).wait()
        pltpu.sync_copy(rows_v, acc_sh.at[dst_v.at[i]], add=True)
        return carry

    lax.fori_loop(0, NCHUNK, body, 0)
    plsc.subcore_barrier()
    pltpu.sync_copy(acc_sh.at[pl.ds(s * RPT, RPT)],
                    out_hbm.at[c, pl.ds(s * RPT, RPT)])


BLK = 1000
GRID = N // BLK

_row_spec = pl.BlockSpec((BLK, D), lambda i: (i, 0))
_w_spec = pl.BlockSpec((D, D), lambda i: (0, 0))
_b_spec = pl.BlockSpec((1, D), lambda i: (0, 0))
_pa_spec = pl.BlockSpec((1, BLK, D), lambda i: (0, i, 0))
_pb_spec = pl.BlockSpec((1, BLK, D), lambda i: (1, i, 0))
_da_spec = pl.BlockSpec((1, BLK, 1), lambda i: (0, i, 0))
_db_spec = pl.BlockSpec((1, BLK, 1), lambda i: (1, i, 0))


def _dis_from(da_ref, db_ref):
    return lax.rsqrt(da_ref[0] + db_ref[0] + 2.0)


def _mm_scale_body(x_ref, w_ref, da_ref, db_ref, h_ref, g_ref):
    h = jnp.dot(x_ref[...], w_ref[...], preferred_element_type=jnp.float32)
    dis = _dis_from(da_ref, db_ref)
    h_ref[...] = h
    g_ref[...] = h * dis


def _layer1_mm(x, W1, degp):
    return pl.pallas_call(
        _mm_scale_body,
        grid=(GRID,),
        in_specs=[_row_spec, _w_spec, _da_spec, _db_spec],
        out_specs=[_row_spec, _row_spec],
        out_shape=[jax.ShapeDtypeStruct((N, D), jnp.float32)] * 2,
    )(x, W1, degp, degp)


def _combine_mm_body(pa_ref, pb_ref, hm_ref, da_ref, db_ref, b_ref, w_ref,
                     h1_ref, hm2_ref, g2_ref):
    dis = _dis_from(da_ref, db_ref)
    acc = pa_ref[0] + pb_ref[0]
    h1 = jnp.maximum(
        acc * dis + hm_ref[...] * (2.0 * dis * dis) + b_ref[...], 0.0)
    hm2 = jnp.dot(h1, w_ref[...], preferred_element_type=jnp.float32)
    h1_ref[...] = h1
    hm2_ref[...] = hm2
    g2_ref[...] = hm2 * dis


def _layer2_mm(p1, h1m, degp, b1, W2):
    return pl.pallas_call(
        _combine_mm_body,
        grid=(GRID,),
        in_specs=[_pa_spec, _pb_spec, _row_spec, _da_spec, _db_spec,
                  _b_spec, _w_spec],
        out_specs=[_row_spec, _row_spec, _row_spec],
        out_shape=[jax.ShapeDtypeStruct((N, D), jnp.float32)] * 3,
    )(p1, p1, h1m, degp, degp, b1, W2)


def _final_body(pa_ref, pb_ref, hm_ref, da_ref, db_ref, b_ref, out_ref):
    dis = _dis_from(da_ref, db_ref)
    acc = pa_ref[0] + pb_ref[0]
    out_ref[...] = acc * dis + hm_ref[...] * (2.0 * dis * dis) + b_ref[...]


def _final_combine(p2, h2m, degp, b2):
    return pl.pallas_call(
        _final_body,
        grid=(GRID,),
        in_specs=[_pa_spec, _pb_spec, _row_spec, _da_spec, _db_spec,
                  _b_spec],
        out_specs=_row_spec,
        out_shape=jax.ShapeDtypeStruct((N, D), jnp.float32),
    )(p2, p2, h2m, degp, degp, b2)


def kernel(x, edge_index, W1, b1, W2, b2):
    src = edge_index[0].reshape(NW, NCHUNK, CK)
    dst = edge_index[1].reshape(NW, NCHUNK, CK)
    zeros128 = jnp.zeros((N_PAD, D), jnp.float32)
    zeros16 = jnp.zeros((N_PAD, 16), jnp.float32)

    degp = _deg_count(dst, zeros16)
    degcol = degp[:, :, :1]

    h1m, g1 = _layer1_mm(x, W1, degcol)
    p1 = _edge_agg(g1, src, dst, zeros128)
    h1, h2m, g2 = _layer2_mm(p1, h1m, degcol, b1.reshape(1, D), W2)
    p2 = _edge_agg(g2, src, dst, zeros128)
    out = _final_combine(p2, h2m, degcol, b2.reshape(1, D))
    return (out, h1)
